# Initial kernel scaffold; baseline (speedup 1.0000x reference)
import jax
import jax.numpy as jnp
from jax.experimental import pallas as pl


def kernel(x, edge_index, W, b, prelu_weight):
    raise NotImplementedError("write your pallas kernel here")


# speedup vs baseline: 11.5126x; 11.5126x over previous
import functools

import jax
import jax.numpy as jnp
from jax import lax
from jax.experimental import pallas as pl
from jax.experimental.pallas import tpu as pltpu
from jax.experimental.pallas import tpu_sc as plsc

N = 10000
D = 128
E = 320000

NC = 2
NS = 16
NW = NC * NS

NPAD = 10240
ROWS_PER_TILE = NPAD // NS
SINK = NPAD - 1

CH = 128
EPAD = 327680
NCHUNK = EPAD // (NW * CH)

_mesh = plsc.VectorSubcoreMesh(
    core_axis_name="c", subcore_axis_name="s", num_cores=NC, num_subcores=NS
)


@functools.partial(
    pl.kernel,
    out_type=jax.ShapeDtypeStruct((NC, NPAD), jnp.float32),
    mesh=_mesh,
    scratch_types=[
        pltpu.VMEM((NCHUNK, CH), jnp.int32),
        pltpu.VMEM((CH,), jnp.float32),
        pltpu.VMEM_SHARED((NPAD,), jnp.float32),
    ],
)
def _deg_kernel(dst_hbm, ones_hbm, zeros_hbm, out_hbm, idx_v, ones_v, deg_sh):
    c = lax.axis_index("c")
    s = lax.axis_index("s")
    wid = s * NC + c
    row0 = s * ROWS_PER_TILE
    pltpu.sync_copy(zeros_hbm, deg_sh.at[pl.ds(row0, ROWS_PER_TILE)])
    pltpu.sync_copy(ones_hbm, ones_v)
    pltpu.sync_copy(dst_hbm.at[wid], idx_v)
    plsc.subcore_barrier()

    def body(j, carry):
        pltpu.sync_copy(ones_v, deg_sh.at[idx_v.at[j]], add=True)
        return carry

    lax.fori_loop(0, NCHUNK, body, 0)
    plsc.subcore_barrier()
    pltpu.sync_copy(
        deg_sh.at[pl.ds(row0, ROWS_PER_TILE)],
        out_hbm.at[c, pl.ds(row0, ROWS_PER_TILE)],
    )


@functools.partial(
    pl.kernel,
    out_type=jax.ShapeDtypeStruct((NC, NPAD, D), jnp.float32),
    mesh=_mesh,
    scratch_types=[
        pltpu.VMEM((NCHUNK, CH), jnp.int32),
        pltpu.VMEM((NCHUNK, CH), jnp.int32),
        pltpu.VMEM((CH, D), jnp.float32),
        pltpu.VMEM_SHARED((NPAD, D), jnp.float32),
        pltpu.SemaphoreType.DMA,
    ],
)
def _edge_kernel(g_hbm, src_hbm, dst_hbm, zeros_hbm, out_hbm,
                 src_v, dst_v, buf_v, acc_sh, sem):
    c = lax.axis_index("c")
    s = lax.axis_index("s")
    wid = s * NC + c
    row0 = s * ROWS_PER_TILE
    pltpu.sync_copy(zeros_hbm, acc_sh.at[pl.ds(row0, ROWS_PER_TILE)])
    pltpu.sync_copy(src_hbm.at[wid], src_v)
    pltpu.sync_copy(dst_hbm.at[wid], dst_v)
    plsc.subcore_barrier()

    def body(j, carry):
        pltpu.async_copy(g_hbm.at[src_v.at[j]], buf_v, sem).wait()
        pltpu.sync_copy(buf_v, acc_sh.at[dst_v.at[j]], add=True)
        return carry

    lax.fori_loop(0, NCHUNK, body, 0)
    plsc.subcore_barrier()
    pltpu.sync_copy(
        acc_sh.at[pl.ds(row0, ROWS_PER_TILE)],
        out_hbm.at[c, pl.ds(row0, ROWS_PER_TILE)],
    )


def _dinv_block(deg_ref):
    deg = deg_ref[0, :] + deg_ref[1, :] + 1.0
    return lax.rsqrt(jnp.maximum(deg, 1e-12))


def _mm_body(deg_ref, x_ref, wt_ref, g_ref):
    dinv = _dinv_block(deg_ref)
    h = jnp.dot(x_ref[...], wt_ref[...], preferred_element_type=jnp.float32)
    g_ref[...] = h * dinv[:, None]


def _fin_body(w_ref, deg_ref, acc_ref, g_ref, b_ref, o_ref):
    dinv = _dinv_block(deg_ref)
    z = (acc_ref[0] + acc_ref[1] + g_ref[...]) * dinv[:, None] + b_ref[...]
    o_ref[...] = jnp.where(z >= 0, z, w_ref[0] * z)


_BR = 256


def kernel(x, edge_index, W, b, prelu_weight):
    src = edge_index[0].astype(jnp.int32)
    dst = edge_index[1].astype(jnp.int32)
    pad = jnp.full((EPAD - E,), SINK, jnp.int32)
    src_t = jnp.concatenate([src, pad]).reshape(NW, NCHUNK, CH)
    dst_t = jnp.concatenate([dst, pad]).reshape(NW, NCHUNK, CH)
    x_pad = jnp.zeros((NPAD, D), jnp.float32).at[:N].set(x)
    wt = W.T

    ones1 = jnp.ones((CH,), jnp.float32)
    zeros1 = jnp.zeros((ROWS_PER_TILE,), jnp.float32)
    zerosd = jnp.zeros((ROWS_PER_TILE, D), jnp.float32)

    degp = _deg_kernel(dst_t, ones1, zeros1)

    g = pl.pallas_call(
        _mm_body,
        grid=(NPAD // _BR,),
        in_specs=[
            pl.BlockSpec((NC, _BR), lambda i: (0, i)),
            pl.BlockSpec((_BR, D), lambda i: (i, 0)),
            pl.BlockSpec((D, D), lambda i: (0, 0)),
        ],
        out_specs=pl.BlockSpec((_BR, D), lambda i: (i, 0)),
        out_shape=jax.ShapeDtypeStruct((NPAD, D), jnp.float32),
    )(degp, x_pad, wt)

    accp = _edge_kernel(g, src_t, dst_t, zerosd)

    out = pl.pallas_call(
        _fin_body,
        grid=(NPAD // _BR,),
        in_specs=[
            pl.BlockSpec(memory_space=pltpu.SMEM),
            pl.BlockSpec((NC, _BR), lambda i: (0, i)),
            pl.BlockSpec((NC, _BR, D), lambda i: (0, i, 0)),
            pl.BlockSpec((_BR, D), lambda i: (i, 0)),
            pl.BlockSpec((1, D), lambda i: (0, 0)),
        ],
        out_specs=pl.BlockSpec((_BR, D), lambda i: (i, 0)),
        out_shape=jax.ShapeDtypeStruct((NPAD, D), jnp.float32),
    )(prelu_weight.reshape(1), degp, accp, g, b.reshape(1, D))

    return out[:N]


# speedup vs baseline: 12.5532x; 1.0904x over previous
import functools

import jax
import jax.numpy as jnp
from jax import lax
from jax.experimental import pallas as pl
from jax.experimental.pallas import tpu as pltpu
from jax.experimental.pallas import tpu_sc as plsc

N = 10000
D = 128
E = 320000

NC = 2
NS = 16
NW = NC * NS

NPAD = 10240
ROWS_PER_TILE = NPAD // NS
SINK = NPAD - 1

CH = 128
EPAD = 327680
NCHUNK = EPAD // (NW * CH)

_mesh = plsc.VectorSubcoreMesh(
    core_axis_name="c", subcore_axis_name="s", num_cores=NC, num_subcores=NS
)


@functools.partial(
    pl.kernel,
    out_type=jax.ShapeDtypeStruct((NC, NPAD), jnp.float32),
    mesh=_mesh,
    scratch_types=[
        pltpu.VMEM((NCHUNK, CH), jnp.int32),
        pltpu.VMEM((CH,), jnp.float32),
        pltpu.VMEM_SHARED((NPAD,), jnp.float32),
    ],
)
def _deg_kernel(dst_hbm, ones_hbm, zeros_hbm, out_hbm, idx_v, ones_v, deg_sh):
    c = lax.axis_index("c")
    s = lax.axis_index("s")
    wid = s * NC + c
    row0 = s * ROWS_PER_TILE
    pltpu.sync_copy(zeros_hbm, deg_sh.at[pl.ds(row0, ROWS_PER_TILE)])
    pltpu.sync_copy(ones_hbm, ones_v)
    pltpu.sync_copy(dst_hbm.at[wid], idx_v)
    plsc.subcore_barrier()

    def body(j, carry):
        pltpu.sync_copy(ones_v, deg_sh.at[idx_v.at[j]], add=True)
        return carry

    lax.fori_loop(0, NCHUNK, body, 0)
    plsc.subcore_barrier()
    pltpu.sync_copy(
        deg_sh.at[pl.ds(row0, ROWS_PER_TILE)],
        out_hbm.at[c, pl.ds(row0, ROWS_PER_TILE)],
    )


NPHASE = 2
PCHUNK = NCHUNK // NPHASE


@functools.partial(
    pl.kernel,
    out_type=jax.ShapeDtypeStruct((NC, NPAD, D), jnp.float32),
    mesh=_mesh,
    scratch_types=[
        pltpu.VMEM((PCHUNK, CH), jnp.int32),
        pltpu.VMEM((PCHUNK, CH), jnp.int32),
        pltpu.VMEM((CH, D), jnp.float32),
        pltpu.VMEM((CH, D), jnp.float32),
        pltpu.VMEM_SHARED((NPAD, D), jnp.float32),
        pltpu.SemaphoreType.DMA,
        pltpu.SemaphoreType.DMA,
    ],
)
def _edge_kernel(g_hbm, src_hbm, dst_hbm, zeros_hbm, out_hbm,
                 src_v, dst_v, buf0_v, buf1_v, acc_sh, sem0, sem1):
    c = lax.axis_index("c")
    s = lax.axis_index("s")
    wid = s * NC + c
    row0 = s * ROWS_PER_TILE
    pltpu.sync_copy(zeros_hbm, acc_sh.at[pl.ds(row0, ROWS_PER_TILE)])

    bufs = (buf0_v, buf1_v)
    sems = (sem0, sem1)

    def gather(j, b):
        pltpu.async_copy(g_hbm.at[src_v.at[j]], bufs[b], sems[b])

    def drain_scatter(j, b):
        pltpu.make_async_copy(g_hbm.at[src_v.at[j]], bufs[b], sems[b]).wait()
        pltpu.sync_copy(bufs[b], acc_sh.at[dst_v.at[j]], add=True)

    def load_idx(p):
        pltpu.sync_copy(src_hbm.at[wid, pl.ds(p * PCHUNK, PCHUNK)], src_v)
        pltpu.sync_copy(dst_hbm.at[wid, pl.ds(p * PCHUNK, PCHUNK)], dst_v)

    def run_phase():
        gather(0, 0)

        def body(i, carry):
            j = i * 2
            gather(j + 1, 1)
            drain_scatter(j, 0)

            @pl.when(j + 2 < PCHUNK)
            def _():
                gather(j + 2, 0)

            drain_scatter(j + 1, 1)
            return carry

        lax.fori_loop(0, PCHUNK // 2, body, 0)

    load_idx(0)
    plsc.subcore_barrier()
    run_phase()
    load_idx(1)
    run_phase()
    plsc.subcore_barrier()
    pltpu.sync_copy(
        acc_sh.at[pl.ds(row0, ROWS_PER_TILE)],
        out_hbm.at[c, pl.ds(row0, ROWS_PER_TILE)],
    )


def _dinv_block(deg_ref):
    deg = deg_ref[0, :] + deg_ref[1, :] + 1.0
    return lax.rsqrt(jnp.maximum(deg, 1e-12))


def _mm_body(deg_ref, x_ref, wt_ref, g_ref):
    dinv = _dinv_block(deg_ref)
    h = jnp.dot(x_ref[...], wt_ref[...], preferred_element_type=jnp.float32)
    g_ref[...] = h * dinv[:, None]


def _fin_body(w_ref, deg_ref, acc_ref, g_ref, b_ref, o_ref):
    dinv = _dinv_block(deg_ref)
    z = (acc_ref[0] + acc_ref[1] + g_ref[...]) * dinv[:, None] + b_ref[...]
    o_ref[...] = jnp.where(z >= 0, z, w_ref[0] * z)


_BR = 256


def kernel(x, edge_index, W, b, prelu_weight):
    src = edge_index[0].astype(jnp.int32)
    dst = edge_index[1].astype(jnp.int32)
    pad = jnp.full((EPAD - E,), SINK, jnp.int32)
    src_t = jnp.concatenate([src, pad]).reshape(NW, NCHUNK, CH)
    dst_t = jnp.concatenate([dst, pad]).reshape(NW, NCHUNK, CH)
    x_pad = jnp.zeros((NPAD, D), jnp.float32).at[:N].set(x)
    wt = W.T

    ones1 = jnp.ones((CH,), jnp.float32)
    zeros1 = jnp.zeros((ROWS_PER_TILE,), jnp.float32)
    zerosd = jnp.zeros((ROWS_PER_TILE, D), jnp.float32)

    degp = _deg_kernel(dst_t, ones1, zeros1)

    g = pl.pallas_call(
        _mm_body,
        grid=(NPAD // _BR,),
        in_specs=[
            pl.BlockSpec((NC, _BR), lambda i: (0, i)),
            pl.BlockSpec((_BR, D), lambda i: (i, 0)),
            pl.BlockSpec((D, D), lambda i: (0, 0)),
        ],
        out_specs=pl.BlockSpec((_BR, D), lambda i: (i, 0)),
        out_shape=jax.ShapeDtypeStruct((NPAD, D), jnp.float32),
    )(degp, x_pad, wt)

    accp = _edge_kernel(g, src_t, dst_t, zerosd)

    out = pl.pallas_call(
        _fin_body,
        grid=(NPAD // _BR,),
        in_specs=[
            pl.BlockSpec(memory_space=pltpu.SMEM),
            pl.BlockSpec((NC, _BR), lambda i: (0, i)),
            pl.BlockSpec((NC, _BR, D), lambda i: (0, i, 0)),
            pl.BlockSpec((_BR, D), lambda i: (i, 0)),
            pl.BlockSpec((1, D), lambda i: (0, 0)),
        ],
        out_specs=pl.BlockSpec((_BR, D), lambda i: (i, 0)),
        out_shape=jax.ShapeDtypeStruct((NPAD, D), jnp.float32),
    )(prelu_weight.reshape(1), degp, accp, g, b.reshape(1, D))

    return out[:N]


# speedup vs baseline: 36.4549x; 2.9040x over previous
import functools

import jax
import jax.numpy as jnp
from jax import lax
from jax.experimental import pallas as pl
from jax.experimental.pallas import tpu as pltpu
from jax.experimental.pallas import tpu_sc as plsc

N = 10000
D = 128
E = 320000

NC = 2
NS = 16
NW = NC * NS

NPAD = 10240
ROWS_PER_TILE = NPAD // NS
SINK = NPAD - 1

CH = 128
EPAD = 327680
NCHUNK = EPAD // (NW * CH)

_mesh = plsc.VectorSubcoreMesh(
    core_axis_name="c", subcore_axis_name="s", num_cores=NC, num_subcores=NS
)


@functools.partial(
    pl.kernel,
    out_type=jax.ShapeDtypeStruct((NC, NPAD), jnp.float32),
    mesh=_mesh,
    scratch_types=[
        pltpu.VMEM((NCHUNK, CH), jnp.int32),
        pltpu.VMEM((CH,), jnp.float32),
        pltpu.VMEM_SHARED((NPAD,), jnp.float32),
    ],
)
def _deg_kernel(dst_hbm, ones_hbm, zeros_hbm, out_hbm, idx_v, ones_v, deg_sh):
    c = lax.axis_index("c")
    s = lax.axis_index("s")
    wid = s * NC + c
    row0 = s * ROWS_PER_TILE
    pltpu.sync_copy(zeros_hbm, deg_sh.at[pl.ds(row0, ROWS_PER_TILE)])
    pltpu.sync_copy(ones_hbm, ones_v)
    pltpu.sync_copy(dst_hbm.at[wid], idx_v)
    plsc.subcore_barrier()

    def body(j, carry):
        pltpu.sync_copy(ones_v, deg_sh.at[idx_v.at[j]], add=True)
        return carry

    lax.fori_loop(0, NCHUNK, body, 0)
    plsc.subcore_barrier()
    pltpu.sync_copy(
        deg_sh.at[pl.ds(row0, ROWS_PER_TILE)],
        out_hbm.at[c, pl.ds(row0, ROWS_PER_TILE)],
    )


NPHASE = 2
PCHUNK = NCHUNK // NPHASE


@functools.partial(
    pl.kernel,
    out_type=jax.ShapeDtypeStruct((NC, NPAD, D), jnp.float32),
    mesh=_mesh,
    scratch_types=[
        pltpu.VMEM((PCHUNK, CH), jnp.int32),
        pltpu.VMEM((PCHUNK, CH), jnp.int32),
        pltpu.VMEM((CH, D), jnp.float32),
        pltpu.VMEM((CH, D), jnp.float32),
        pltpu.VMEM_SHARED((NPAD, D), jnp.float32),
        pltpu.SemaphoreType.DMA,
        pltpu.SemaphoreType.DMA,
    ],
)
def _edge_kernel(g_hbm, src_hbm, dst_hbm, zeros_hbm, out_hbm,
                 src_v, dst_v, buf0_v, buf1_v, acc_sh, sem0, sem1):
    c = lax.axis_index("c")
    s = lax.axis_index("s")
    wid = s * NC + c
    row0 = s * ROWS_PER_TILE
    pltpu.sync_copy(zeros_hbm, acc_sh.at[pl.ds(row0, ROWS_PER_TILE)])

    bufs = (buf0_v, buf1_v)
    sems = (sem0, sem1)

    def gather(j, b):
        pltpu.async_copy(g_hbm.at[src_v.at[j]], bufs[b], sems[b])

    def drain_scatter(j, b):
        pltpu.make_async_copy(g_hbm.at[src_v.at[j]], bufs[b], sems[b]).wait()
        pltpu.sync_copy(bufs[b], acc_sh.at[dst_v.at[j]], add=True)

    def load_idx(p):
        pltpu.sync_copy(src_hbm.at[wid, pl.ds(p * PCHUNK, PCHUNK)], src_v)
        pltpu.sync_copy(dst_hbm.at[wid, pl.ds(p * PCHUNK, PCHUNK)], dst_v)

    def run_phase():
        gather(0, 0)

        def body(i, carry):
            j = i * 2
            gather(j + 1, 1)
            drain_scatter(j, 0)

            @pl.when(j + 2 < PCHUNK)
            def _():
                gather(j + 2, 0)

            drain_scatter(j + 1, 1)
            return carry

        lax.fori_loop(0, PCHUNK // 2, body, 0)

    load_idx(0)
    plsc.subcore_barrier()
    run_phase()
    load_idx(1)
    run_phase()
    plsc.subcore_barrier()
    pltpu.sync_copy(
        acc_sh.at[pl.ds(row0, ROWS_PER_TILE)],
        out_hbm.at[c, pl.ds(row0, ROWS_PER_TILE)],
    )


def _dinv_block(deg_ref):
    deg = deg_ref[0, :] + deg_ref[1, :] + 1.0
    return lax.rsqrt(jnp.maximum(deg, 1e-12))


def _mm_body(deg_ref, x_ref, wt_ref, g_ref):
    dinv = _dinv_block(deg_ref)
    h = jnp.dot(x_ref[...], wt_ref[...], preferred_element_type=jnp.float32)
    g_ref[...] = h * dinv[:, None]


def _fin_body(w_ref, deg_ref, acc_ref, g_ref, b_ref, o_ref):
    dinv = _dinv_block(deg_ref)
    z = (acc_ref[0] + acc_ref[1] + g_ref[...]) * dinv[:, None] + b_ref[...]
    o_ref[...] = jnp.where(z >= 0, z, w_ref[0] * z)


_BR = 256


def kernel(x, edge_index, W, b, prelu_weight):
    src = edge_index[0].astype(jnp.int32)
    dst = edge_index[1].astype(jnp.int32)
    pad = N + jnp.arange(EPAD - E, dtype=jnp.int32) % (NPAD - N)
    src_t = jnp.concatenate([src, pad]).reshape(NW, NCHUNK, CH)
    dst_t = jnp.concatenate([dst, pad]).reshape(NW, NCHUNK, CH)
    x_pad = jnp.zeros((NPAD, D), jnp.float32).at[:N].set(x)
    wt = W.T

    ones1 = jnp.ones((CH,), jnp.float32)
    zeros1 = jnp.zeros((ROWS_PER_TILE,), jnp.float32)
    zerosd = jnp.zeros((ROWS_PER_TILE, D), jnp.float32)

    degp = _deg_kernel(dst_t, ones1, zeros1)

    g = pl.pallas_call(
        _mm_body,
        grid=(NPAD // _BR,),
        in_specs=[
            pl.BlockSpec((NC, _BR), lambda i: (0, i)),
            pl.BlockSpec((_BR, D), lambda i: (i, 0)),
            pl.BlockSpec((D, D), lambda i: (0, 0)),
        ],
        out_specs=pl.BlockSpec((_BR, D), lambda i: (i, 0)),
        out_shape=jax.ShapeDtypeStruct((NPAD, D), jnp.float32),
    )(degp, x_pad, wt)

    accp = _edge_kernel(g, src_t, dst_t, zerosd)

    out = pl.pallas_call(
        _fin_body,
        grid=(NPAD // _BR,),
        in_specs=[
            pl.BlockSpec(memory_space=pltpu.SMEM),
            pl.BlockSpec((NC, _BR), lambda i: (0, i)),
            pl.BlockSpec((NC, _BR, D), lambda i: (0, i, 0)),
            pl.BlockSpec((_BR, D), lambda i: (i, 0)),
            pl.BlockSpec((1, D), lambda i: (0, 0)),
        ],
        out_specs=pl.BlockSpec((_BR, D), lambda i: (i, 0)),
        out_shape=jax.ShapeDtypeStruct((NPAD, D), jnp.float32),
    )(prelu_weight.reshape(1), degp, accp, g, b.reshape(1, D))

    return out[:N]


# speedup vs baseline: 38.5844x; 1.0584x over previous
import functools

import jax
import jax.numpy as jnp
from jax import lax
from jax.experimental import pallas as pl
from jax.experimental.pallas import tpu as pltpu
from jax.experimental.pallas import tpu_sc as plsc

N = 10000
D = 128
E = 320000

NC = 2
NS = 16
NW = NC * NS

NPAD = 10240
ROWS_PER_TILE = NPAD // NS

CH = 125
CPT = 80
PCH = 40

_mesh = plsc.VectorSubcoreMesh(
    core_axis_name="c", subcore_axis_name="s", num_cores=NC, num_subcores=NS
)


@functools.partial(
    pl.kernel,
    out_type=jax.ShapeDtypeStruct((NC, NPAD), jnp.float32),
    mesh=_mesh,
    scratch_types=[
        pltpu.VMEM((CPT, CH), jnp.int32),
        pltpu.VMEM((CH,), jnp.float32),
        pltpu.VMEM_SHARED((NPAD,), jnp.float32),
    ],
)
def _deg_kernel(dst_hbm, ones_hbm, zeros_hbm, out_hbm, idx_v, ones_v, deg_sh):
    c = lax.axis_index("c")
    s = lax.axis_index("s")
    wid = s * NC + c
    row0 = s * ROWS_PER_TILE
    pltpu.sync_copy(zeros_hbm, deg_sh.at[pl.ds(row0, ROWS_PER_TILE)])
    pltpu.sync_copy(ones_hbm, ones_v)
    pltpu.sync_copy(dst_hbm.at[wid], idx_v)
    plsc.subcore_barrier()

    def body(j, carry):
        pltpu.sync_copy(ones_v, deg_sh.at[idx_v.at[j]], add=True)
        return carry

    lax.fori_loop(0, CPT, body, 0)
    plsc.subcore_barrier()
    pltpu.sync_copy(deg_sh.at[pl.ds(row0, ROWS_PER_TILE)],
                    out_hbm.at[c, pl.ds(row0, ROWS_PER_TILE)])


@functools.partial(
    pl.kernel,
    out_type=jax.ShapeDtypeStruct((NC, NPAD, D), jnp.float32),
    mesh=_mesh,
    scratch_types=[
        pltpu.VMEM((PCH, CH), jnp.int32),
        pltpu.VMEM((PCH, CH), jnp.int32),
        pltpu.VMEM((CH, D), jnp.float32),
        pltpu.VMEM((CH, D), jnp.float32),
        pltpu.VMEM_SHARED((NPAD, D), jnp.float32),
        pltpu.SemaphoreType.DMA,
        pltpu.SemaphoreType.DMA,
    ],
)
def _edge_kernel(g_hbm, src_hbm, dst_hbm, zeros_hbm, out_hbm,
                 src_v, dst_v, buf0_v, buf1_v, acc_sh, sem0, sem1):
    c = lax.axis_index("c")
    s = lax.axis_index("s")
    wid = s * NC + c
    row0 = s * ROWS_PER_TILE

    bufs = (buf0_v, buf1_v)
    sems = (sem0, sem1)

    def gather(j, b):
        pltpu.async_copy(g_hbm.at[src_v.at[j]], bufs[b], sems[b])

    def drain_scatter(j, b):
        pltpu.make_async_copy(g_hbm.at[src_v.at[j]], bufs[b], sems[b]).wait()
        pltpu.sync_copy(bufs[b], acc_sh.at[dst_v.at[j]], add=True)

    def load_idx(p):
        pltpu.sync_copy(src_hbm.at[wid, pl.ds(p * PCH, PCH)], src_v)
        pltpu.sync_copy(dst_hbm.at[wid, pl.ds(p * PCH, PCH)], dst_v)

    def run_phase():
        gather(0, 0)

        def body(i, carry):
            j = i * 2
            gather(j + 1, 1)
            drain_scatter(j, 0)

            @pl.when(j + 2 < PCH)
            def _():
                gather(j + 2, 0)

            drain_scatter(j + 1, 1)
            return carry

        lax.fori_loop(0, PCH // 2, body, 0)

    load_idx(0)
    pltpu.sync_copy(zeros_hbm, acc_sh.at[pl.ds(row0, ROWS_PER_TILE)])
    plsc.subcore_barrier()
    run_phase()
    load_idx(1)
    run_phase()
    plsc.subcore_barrier()
    pltpu.sync_copy(acc_sh.at[pl.ds(row0, ROWS_PER_TILE)],
                    out_hbm.at[c, pl.ds(row0, ROWS_PER_TILE)])


_BR = 256


def _dinv_block(deg_ref):
    i = pl.program_id(0)
    sl = pl.ds(i * _BR, _BR)
    deg = deg_ref[0, sl] + deg_ref[1, sl] + 1.0
    return lax.rsqrt(jnp.maximum(deg, 1e-12))


def _mm_body(deg_ref, x_ref, wt_ref, g_ref):
    dinv = _dinv_block(deg_ref)
    h = jnp.dot(x_ref[...], wt_ref[...], preferred_element_type=jnp.float32)
    g_ref[...] = h * dinv[:, None]


def _fin_body(w_ref, deg_ref, acc_ref, g_ref, b_ref, o_ref):
    dinv = _dinv_block(deg_ref)
    z = (acc_ref[0] + acc_ref[1] + g_ref[...]) * dinv[:, None] + b_ref[...]
    o_ref[...] = jnp.where(z >= 0, z, w_ref[0] * z)


def kernel(x, edge_index, W, b, prelu_weight):
    ei = edge_index.astype(jnp.int32).reshape(2, NW, CPT, CH)
    src_t = ei[0]
    dst_t = ei[1]
    wt = W.T

    ones1 = jnp.ones((CH,), jnp.float32)
    zeros1 = jnp.zeros((ROWS_PER_TILE,), jnp.float32)
    zerosd = jnp.zeros((ROWS_PER_TILE, D), jnp.float32)

    degp = _deg_kernel(dst_t, ones1, zeros1)

    g = pl.pallas_call(
        _mm_body,
        grid=(pl.cdiv(N, _BR),),
        in_specs=[
            pl.BlockSpec((NC, NPAD), lambda i: (0, 0)),
            pl.BlockSpec((_BR, D), lambda i: (i, 0)),
            pl.BlockSpec((D, D), lambda i: (0, 0)),
        ],
        out_specs=pl.BlockSpec((_BR, D), lambda i: (i, 0)),
        out_shape=jax.ShapeDtypeStruct((N, D), jnp.float32),
    )(degp, x, wt)

    accp = _edge_kernel(g, src_t, dst_t, zerosd)

    out = pl.pallas_call(
        _fin_body,
        grid=(pl.cdiv(N, _BR),),
        in_specs=[
            pl.BlockSpec(memory_space=pltpu.SMEM),
            pl.BlockSpec((NC, NPAD), lambda i: (0, 0)),
            pl.BlockSpec((NC, _BR, D), lambda i: (0, i, 0)),
            pl.BlockSpec((_BR, D), lambda i: (i, 0)),
            pl.BlockSpec((1, D), lambda i: (0, 0)),
        ],
        out_specs=pl.BlockSpec((_BR, D), lambda i: (i, 0)),
        out_shape=jax.ShapeDtypeStruct((N, D), jnp.float32),
    )(prelu_weight.reshape(1), degp, accp, g, b.reshape(1, D))

    return out
